# baseline (speedup 1.0000x reference)
import jax
import jax.numpy as jnp
from jax import lax
from jax.experimental import pallas as pl
from jax.experimental.pallas import tpu as pltpu

_B, _N, _Q = 16, 2048, 128
_CHUNK = 256
_K = _N // _CHUNK


def _fused_body(xs_ref, ys_ref, s_ref, wt_ref, b_ref, out_ref, acc_ref):
    k = pl.program_id(1)
    xc = xs_ref[0, 0]
    yc = ys_ref[0, 0]
    sr = s_ref[...]
    phi = jnp.maximum(jnp.minimum(sr - xc, yc - sr), 0.0)
    part = jnp.sum(phi, axis=0, keepdims=True)

    @pl.when(k == 0)
    def _():
        acc_ref[...] = part

    @pl.when(k > 0)
    def _():
        acc_ref[...] = acc_ref[...] + part

    @pl.when(k == _K - 1)
    def _():
        acc = lax.dot_general(
            acc_ref[...], wt_ref[...], (((1,), (0,)), ((), ())),
            preferred_element_type=jnp.float32,
        )
        out_ref[0] = jnp.maximum(acc + b_ref[...], 0.0)


_fused = pl.pallas_call(
    _fused_body,
    grid=(_B, _K),
    in_specs=[
        pl.BlockSpec((1, 1, _CHUNK, 1), lambda b, k: (b, k, 0, 0)),
        pl.BlockSpec((1, 1, _CHUNK, 1), lambda b, k: (b, k, 0, 0)),
        pl.BlockSpec((1, _Q), lambda b, k: (0, 0)),
        pl.BlockSpec((_Q, _Q), lambda b, k: (0, 0)),
        pl.BlockSpec((1, _Q), lambda b, k: (0, 0)),
    ],
    out_specs=pl.BlockSpec((1, 1, _Q), lambda b, k: (b, 0, 0)),
    out_shape=jax.ShapeDtypeStruct((_B, 1, _Q), jnp.float32),
    scratch_shapes=[pltpu.VMEM((1, _Q), jnp.float32)],
)


def kernel(diagram, samples, rho_w, rho_b):
    xs = diagram[:, :, 0].reshape(_B, _K, _CHUNK, 1)
    ys = diagram[:, :, 1].reshape(_B, _K, _CHUNK, 1)
    return _fused(xs, ys, samples.reshape(1, _Q), rho_w.T,
                  rho_b.reshape(1, _Q)).reshape(_B, _Q)


# speedup vs baseline: 6.9536x; 6.9536x over previous
import jax
import jax.numpy as jnp
from jax import lax
from jax.experimental import pallas as pl
from jax.experimental.pallas import tpu as pltpu

_B, _N, _Q = 16, 2048, 128
_CHUNK = 128
_K = _N // _CHUNK


def _fused_body(xs_ref, ys_ref, s_ref, w_ref, b_ref, out_ref):
    sc = s_ref[...]
    xr = xs_ref[0]
    yr = ys_ref[0]
    acc = None
    for k in range(_K):
        xc = lax.slice(xr, (0, k * _CHUNK), (1, (k + 1) * _CHUNK))
        yc = lax.slice(yr, (0, k * _CHUNK), (1, (k + 1) * _CHUNK))
        phi = jnp.maximum(jnp.minimum(sc - xc, yc - sc), 0.0)
        acc = phi if acc is None else acc + phi
    pooled = jnp.sum(acc, axis=1, keepdims=True)
    r = lax.dot_general(
        w_ref[...], pooled, (((1,), (0,)), ((), ())),
        preferred_element_type=jnp.float32,
    )
    out_ref[0] = jnp.maximum(r + b_ref[...], 0.0)


_fused = pl.pallas_call(
    _fused_body,
    grid=(_B,),
    in_specs=[
        pl.BlockSpec((1, 1, _N), lambda b: (b, 0, 0)),
        pl.BlockSpec((1, 1, _N), lambda b: (b, 0, 0)),
        pl.BlockSpec((_Q, 1), lambda b: (0, 0)),
        pl.BlockSpec((_Q, _Q), lambda b: (0, 0)),
        pl.BlockSpec((_Q, 1), lambda b: (0, 0)),
    ],
    out_specs=pl.BlockSpec((1, _Q, 1), lambda b: (b, 0, 0)),
    out_shape=jax.ShapeDtypeStruct((_B, _Q, 1), jnp.float32),
)


def kernel(diagram, samples, rho_w, rho_b):
    xs = diagram[:, :, 0].reshape(_B, 1, _N)
    ys = diagram[:, :, 1].reshape(_B, 1, _N)
    return _fused(xs, ys, samples.reshape(_Q, 1), rho_w,
                  rho_b.reshape(_Q, 1)).reshape(_B, _Q)


# speedup vs baseline: 11.3449x; 1.6315x over previous
import jax
import jax.numpy as jnp
from jax import lax
from jax.experimental import pallas as pl
from jax.experimental.pallas import tpu as pltpu

_B, _N, _Q = 16, 2048, 128
_CHUNK = 128
_K = _N // _CHUNK


def _fused_body(xs_ref, ys_ref, s_ref, w_ref, b_ref, out_ref):
    sc = s_ref[...]
    cols = []
    for b in range(_B):
        acc = None
        for k in range(_K):
            xc = lax.slice(xs_ref[...], (b, k * _CHUNK), (b + 1, (k + 1) * _CHUNK))
            yc = lax.slice(ys_ref[...], (b, k * _CHUNK), (b + 1, (k + 1) * _CHUNK))
            phi = jnp.maximum(jnp.minimum(sc - xc, yc - sc), 0.0)
            acc = phi if acc is None else acc + phi
        cols.append(jnp.sum(acc, axis=1, keepdims=True))
    p = jnp.concatenate(cols, axis=1)
    r = lax.dot_general(
        w_ref[...], p, (((1,), (0,)), ((), ())),
        preferred_element_type=jnp.float32,
    )
    out_ref[...] = jnp.maximum(r + b_ref[...], 0.0)


_fused = pl.pallas_call(
    _fused_body,
    out_shape=jax.ShapeDtypeStruct((_Q, _B), jnp.float32),
)


def kernel(diagram, samples, rho_w, rho_b):
    xs = diagram[:, :, 0]
    ys = diagram[:, :, 1]
    out_t = _fused(xs, ys, samples.reshape(_Q, 1), rho_w,
                   rho_b.reshape(_Q, 1))
    return out_t.T
